# baseline (speedup 1.0000x reference)
import functools

import jax
import jax.numpy as jnp
import numpy as np
from jax import lax
from jax.experimental import pallas as pl
from jax.experimental.pallas import tpu as pltpu
from jax.experimental.pallas import tpu_sc as plsc

N = 10000
E = 320000
D = 128
A = 4
B = 8
H = 8

NUM_WORKERS = 32
CH = 40
N_CHUNKS = E // (NUM_WORKERS * CH)
EDGES_PER_WORKER = CH * N_CHUNKS
N_PAD = 10240
ROWS_PER_SUBCORE = N_PAD // 16
STAGE_ROWS = 40

_INV_SQRT_B = 1.0 / np.sqrt(float(B))
_INV_SQRT_H = 1.0 / np.sqrt(float(H))
_INV_SQRT_D = 1.0 / np.sqrt(float(D))
_POST_SCALE = 1.0 / (np.sqrt(32.0) * np.sqrt(float(D)))
_SC_SCALE = 1.0 / np.sqrt(float(D * A))

_PERM = np.empty((D,), dtype=np.int32)
for _blk in range(D // 32):
    for _half in range(2):
        for _m in range(16):
            _PERM[_blk * 32 + _half * 16 + _m] = _blk * 32 + 2 * _m + _half


def _edge_coef_body(emb_ref, ea_ref, ei_ref, w1_ref, w2e_ref, w2o_ref,
                    out_ref, idx_ref):
    h = jnp.dot(emb_ref[...], w1_ref[...],
                preferred_element_type=jnp.float32) * _INV_SQRT_B
    h = h * jax.nn.sigmoid(h)
    ea = ea_ref[...] * _INV_SQRT_H
    we = jnp.dot(h, w2e_ref[...], preferred_element_type=jnp.float32) * ea
    wo = jnp.dot(h, w2o_ref[...], preferred_element_type=jnp.float32) * ea
    pe = lax.convert_element_type(
        lax.bitcast_convert_type(we.astype(jnp.bfloat16), jnp.uint16),
        jnp.int32)
    po = lax.convert_element_type(
        lax.bitcast_convert_type(wo.astype(jnp.bfloat16), jnp.uint16),
        jnp.int32)
    out_ref[...] = jnp.bitwise_or(pe, lax.shift_left(po, 16))
    idx_ref[...] = ei_ref[...].reshape(2, _EBLK // CH, CH).swapaxes(0, 1)


_EBLK = 2560


def _edge_coef(emb, ea, ei, w1, w2e, w2o):
    grid = E // _EBLK
    return pl.pallas_call(
        _edge_coef_body,
        grid=(grid,),
        in_specs=[
            pl.BlockSpec((_EBLK, B), lambda i: (i, 0)),
            pl.BlockSpec((_EBLK, 1), lambda i: (i, 0)),
            pl.BlockSpec((2, _EBLK), lambda i: (0, i)),
            pl.BlockSpec((B, H), lambda i: (0, 0)),
            pl.BlockSpec((H, D // 2), lambda i: (0, 0)),
            pl.BlockSpec((H, D // 2), lambda i: (0, 0)),
        ],
        out_specs=[
            pl.BlockSpec((_EBLK, D // 2), lambda i: (i, 0)),
            pl.BlockSpec((_EBLK // CH, 2, CH), lambda i: (i, 0, 0)),
        ],
        out_shape=[
            jax.ShapeDtypeStruct((E, D // 2), jnp.int32),
            jax.ShapeDtypeStruct((E // CH, 2, CH), jnp.int32),
        ],
    )(emb, ea, ei, w1, w2e, w2o)


def _lin1_body(x_ref, w_ref, o_ref):
    o_ref[...] = jnp.dot(x_ref[...], w_ref[...],
                         preferred_element_type=jnp.float32) * _INV_SQRT_D


def _lin1(x, w):
    blk = 2000
    return pl.pallas_call(
        _lin1_body,
        grid=(N // blk,),
        in_specs=[
            pl.BlockSpec((blk, D), lambda i: (i, 0)),
            pl.BlockSpec((D, D), lambda i: (0, 0)),
        ],
        out_specs=pl.BlockSpec((blk, D), lambda i: (i, 0)),
        out_shape=jax.ShapeDtypeStruct((N, D), jnp.float32),
    )(x, w)


def _sc_agg_body(idx_hbm, c_hbm, x1_hbm, out_hbm,
                 idxb, c0, c1, x0, x1b, p0, p1, acc,
                 sem_c0, sem_c1, sem_x0, sem_x1, sem_s0, sem_s1,
                 sem_i0, sem_i1):
    cid = lax.axis_index("c")
    sid = lax.axis_index("s")
    wid = sid * 2 + cid
    chunk0 = wid * N_CHUNKS

    def _zrow(j, carry):
        for k in range(D // 16):
            p0[j, pl.ds(k * 16, 16)] = jnp.zeros((16,), jnp.float32)
        return carry
    lax.fori_loop(0, STAGE_ROWS, _zrow, 0)
    for t in range(ROWS_PER_SUBCORE // STAGE_ROWS):
        start = sid * ROWS_PER_SUBCORE + t * STAGE_ROWS
        pltpu.sync_copy(p0, acc.at[pl.ds(start, STAGE_ROWS)])
    plsc.subcore_barrier()

    def _idx_dst(i):
        return idxb.at[pl.ds(lax.rem(i, 4) * 2, 2)]

    def _load_idx(i, sem_i):
        pltpu.async_copy(idx_hbm.at[chunk0 + i], _idx_dst(i), sem_i)

    def _wait_idx(i, sem_i):
        pltpu.make_async_copy(idx_hbm.at[0], _idx_dst(i), sem_i).wait()

    def _start_cx(i, c_b, x_b, sem_c, sem_x):
        pltpu.async_copy(c_hbm.at[pl.ds((chunk0 + i) * CH, CH)], c_b, sem_c)
        pltpu.async_copy(x1_hbm.at[idxb.at[lax.rem(i, 4) * 2]], x_b, sem_x)

    def _step(i, bufs, obufs):
        c_b, x_b, p_b, sem_c, sem_x, sem_s, sem_i = bufs
        _, _, po_b, _, _, sem_so, _ = obufs
        pltpu.make_async_copy(c_hbm.at[pl.ds(0, CH)], c_b, sem_c).wait()
        pltpu.make_async_copy(x1_hbm.at[idxb.at[0]], x_b, sem_x).wait()

        hi_mask = jnp.full((16,), -65536, jnp.int32)

        @plsc.parallel_loop(0, CH, unroll=4)
        def _row(j):
            for k in range(D // 32):
                cw = c_b[j, pl.ds(k * 16, 16)]
                c_lo = lax.bitcast_convert_type(
                    lax.shift_left(cw, 16), jnp.float32)
                c_hi = lax.bitcast_convert_type(
                    jnp.bitwise_and(cw, hi_mask), jnp.float32)
                p_b[j, pl.ds(k * 32, 16)] = c_lo * x_b[j, pl.ds(k * 32, 16)]
                p_b[j, pl.ds(k * 32 + 16, 16)] = (
                    c_hi * x_b[j, pl.ds(k * 32 + 16, 16)])

        @pl.when(i > 0)
        def _():
            pltpu.make_async_copy(po_b, acc.at[idxb.at[1]], sem_so).wait()
        pltpu.async_copy(p_b, acc.at[idxb.at[lax.rem(i, 4) * 2 + 1]], sem_s,
                         add=True)

        @pl.when(i + 2 < N_CHUNKS)
        def _():
            _wait_idx(i + 2, sem_i)
            _start_cx(i + 2, c_b, x_b, sem_c, sem_x)

        @pl.when(i + 4 < N_CHUNKS)
        def _():
            _load_idx(i + 4, sem_i)

    bufs0 = (c0, x0, p0, sem_c0, sem_x0, sem_s0, sem_i0)
    bufs1 = (c1, x1b, p1, sem_c1, sem_x1, sem_s1, sem_i1)

    pltpu.sync_copy(idx_hbm.at[chunk0], _idx_dst(0))
    pltpu.sync_copy(idx_hbm.at[chunk0 + 1], _idx_dst(1))
    _load_idx(2, sem_i0)
    _load_idx(3, sem_i1)
    _start_cx(0, c0, x0, sem_c0, sem_x0)
    _start_cx(1, c1, x1b, sem_c1, sem_x1)

    def _pair(t, carry):
        i = 2 * t
        _step(i, bufs0, bufs1)
        _step(i + 1, bufs1, bufs0)
        return carry
    lax.fori_loop(0, N_CHUNKS // 2, _pair, 0)
    pltpu.make_async_copy(p1, acc.at[idxb.at[1]], sem_s1).wait()

    plsc.subcore_barrier()
    for t in range(ROWS_PER_SUBCORE // STAGE_ROWS):
        start = sid * ROWS_PER_SUBCORE + t * STAGE_ROWS
        pltpu.sync_copy(acc.at[pl.ds(start, STAGE_ROWS)], p0)
        pltpu.sync_copy(p0, out_hbm.at[pl.ds(cid * N_PAD + start, STAGE_ROWS)])


@functools.partial(
    pl.kernel,
    out_type=jax.ShapeDtypeStruct((2 * N_PAD, D), jnp.float32),
    mesh=plsc.VectorSubcoreMesh(core_axis_name="c", subcore_axis_name="s"),
    scratch_types=[
        pltpu.VMEM((8, CH), jnp.int32),
        pltpu.VMEM((CH, D // 2), jnp.int32),
        pltpu.VMEM((CH, D // 2), jnp.int32),
        pltpu.VMEM((CH, D), jnp.float32),
        pltpu.VMEM((CH, D), jnp.float32),
        pltpu.VMEM((CH, D), jnp.float32),
        pltpu.VMEM((CH, D), jnp.float32),
        pltpu.VMEM_SHARED((N_PAD, D), jnp.float32),
        pltpu.SemaphoreType.DMA,
        pltpu.SemaphoreType.DMA,
        pltpu.SemaphoreType.DMA,
        pltpu.SemaphoreType.DMA,
        pltpu.SemaphoreType.DMA,
        pltpu.SemaphoreType.DMA,
        pltpu.SemaphoreType.DMA,
        pltpu.SemaphoreType.DMA,
    ],
)
def _sc_aggregate(idx_hbm, c_hbm, x1_hbm, out_hbm,
                  idxb, c0, c1, x0, x1b, p0, p1, acc,
                  sem_c0, sem_c1, sem_x0, sem_x1, sem_s0, sem_s1,
                  sem_i0, sem_i1):
    _sc_agg_body(idx_hbm, c_hbm, x1_hbm, out_hbm,
                 idxb, c0, c1, x0, x1b, p0, p1, acc,
                 sem_c0, sem_c1, sem_x0, sem_x1, sem_s0, sem_s1,
                 sem_i0, sem_i1)


def _post_body(p_ref, x_ref, attr_ref, w2_ref, wsc_ref, o_ref):
    agg = p_ref[0] + p_ref[1]
    y = jnp.dot(agg, w2_ref[...],
                preferred_element_type=jnp.float32) * _POST_SCALE
    for v in range(A):
        y = y + jnp.dot(x_ref[...] * attr_ref[:, v:v + 1], wsc_ref[v],
                        preferred_element_type=jnp.float32) * _SC_SCALE
    o_ref[...] = y


def _post(partial, x, attr, w2, wsc_t):
    blk = 2000
    return pl.pallas_call(
        _post_body,
        grid=(N // blk,),
        in_specs=[
            pl.BlockSpec((2, blk, D), lambda i: (0, i, 0)),
            pl.BlockSpec((blk, D), lambda i: (i, 0)),
            pl.BlockSpec((blk, A), lambda i: (i, 0)),
            pl.BlockSpec((D, D), lambda i: (0, 0)),
            pl.BlockSpec((A, D, D), lambda i: (0, 0, 0)),
        ],
        out_specs=pl.BlockSpec((blk, D), lambda i: (i, 0)),
        out_shape=jax.ShapeDtypeStruct((N, D), jnp.float32),
    )(partial, x, attr, w2, wsc_t)


def kernel(node_features, node_attr, edge_attr, edge_embedding, edge_index,
           W_lin1, fc_W1, fc_W2, W_lin2, W_sc):
    perm = jnp.asarray(_PERM)
    c_i32, idx_pack = _edge_coef(edge_embedding, edge_attr, edge_index,
                                 fc_W1, fc_W2[:, 0::2], fc_W2[:, 1::2])
    x1 = _lin1(node_features, W_lin1[:, perm])
    partial = _sc_aggregate(idx_pack, c_i32, x1)
    partial = partial.reshape(2, N_PAD, D)[:, :N, :]
    return _post(partial, node_features, node_attr,
                 W_lin2[perm, :], W_sc.transpose(1, 0, 2))


# speedup vs baseline: 1.0028x; 1.0028x over previous
import functools

import jax
import jax.numpy as jnp
import numpy as np
from jax import lax
from jax.experimental import pallas as pl
from jax.experimental.pallas import tpu as pltpu
from jax.experimental.pallas import tpu_sc as plsc

N = 10000
E = 320000
D = 128
A = 4
B = 8
H = 8

NUM_WORKERS = 32
CH = 40
N_CHUNKS = E // (NUM_WORKERS * CH)
EDGES_PER_WORKER = CH * N_CHUNKS
N_PAD = 10240
ROWS_PER_SUBCORE = N_PAD // 16
STAGE_ROWS = 40

_INV_SQRT_B = 1.0 / np.sqrt(float(B))
_INV_SQRT_H = 1.0 / np.sqrt(float(H))
_INV_SQRT_D = 1.0 / np.sqrt(float(D))
_POST_SCALE = 1.0 / (np.sqrt(32.0) * np.sqrt(float(D)))
_SC_SCALE = 1.0 / np.sqrt(float(D * A))

_PERM = np.empty((D,), dtype=np.int32)
for _blk in range(D // 32):
    for _half in range(2):
        for _m in range(16):
            _PERM[_blk * 32 + _half * 16 + _m] = _blk * 32 + 2 * _m + _half


def _edge_coef_body(emb_ref, ea_ref, ei_ref, w1_ref, w2e_ref, w2o_ref,
                    out_ref, idx_ref):
    h = jnp.dot(emb_ref[...], w1_ref[...],
                preferred_element_type=jnp.float32) * _INV_SQRT_B
    h = h * jax.nn.sigmoid(h)
    h = h * (ea_ref[...] * _INV_SQRT_H)
    we = jnp.dot(h, w2e_ref[...], preferred_element_type=jnp.float32)
    wo = jnp.dot(h, w2o_ref[...], preferred_element_type=jnp.float32)
    pe = lax.convert_element_type(
        lax.bitcast_convert_type(we.astype(jnp.bfloat16), jnp.uint16),
        jnp.int32)
    po = lax.convert_element_type(
        lax.bitcast_convert_type(wo.astype(jnp.bfloat16), jnp.uint16),
        jnp.int32)
    out_ref[...] = jnp.bitwise_or(pe, lax.shift_left(po, 16))
    idx_ref[...] = ei_ref[...].reshape(2, _EBLK // CH, CH).swapaxes(0, 1)


_EBLK = 2560


def _edge_coef(emb, ea, ei, w1, w2e, w2o):
    grid = E // _EBLK
    return pl.pallas_call(
        _edge_coef_body,
        grid=(grid,),
        in_specs=[
            pl.BlockSpec((_EBLK, B), lambda i: (i, 0)),
            pl.BlockSpec((_EBLK, 1), lambda i: (i, 0)),
            pl.BlockSpec((2, _EBLK), lambda i: (0, i)),
            pl.BlockSpec((B, D), lambda i: (0, 0)),
            pl.BlockSpec((D, D // 2), lambda i: (0, 0)),
            pl.BlockSpec((D, D // 2), lambda i: (0, 0)),
        ],
        out_specs=[
            pl.BlockSpec((_EBLK, D // 2), lambda i: (i, 0)),
            pl.BlockSpec((_EBLK // CH, 2, CH), lambda i: (i, 0, 0)),
        ],
        out_shape=[
            jax.ShapeDtypeStruct((E, D // 2), jnp.int32),
            jax.ShapeDtypeStruct((E // CH, 2, CH), jnp.int32),
        ],
    )(emb, ea, ei, w1, w2e, w2o)


def _lin1_body(x_ref, w_ref, o_ref):
    o_ref[...] = jnp.dot(x_ref[...], w_ref[...],
                         preferred_element_type=jnp.float32) * _INV_SQRT_D


def _lin1(x, w):
    blk = 2000
    return pl.pallas_call(
        _lin1_body,
        grid=(N // blk,),
        in_specs=[
            pl.BlockSpec((blk, D), lambda i: (i, 0)),
            pl.BlockSpec((D, D), lambda i: (0, 0)),
        ],
        out_specs=pl.BlockSpec((blk, D), lambda i: (i, 0)),
        out_shape=jax.ShapeDtypeStruct((N, D), jnp.float32),
    )(x, w)


def _sc_agg_body(idx_hbm, c_hbm, x1_hbm, out_hbm,
                 idxb, c0, c1, x0, x1b, p0, p1, acc,
                 sem_c0, sem_c1, sem_x0, sem_x1, sem_s0, sem_s1,
                 sem_i0, sem_i1):
    cid = lax.axis_index("c")
    sid = lax.axis_index("s")
    wid = sid * 2 + cid
    chunk0 = wid * N_CHUNKS

    def _zrow(j, carry):
        for k in range(D // 16):
            p0[j, pl.ds(k * 16, 16)] = jnp.zeros((16,), jnp.float32)
        return carry
    lax.fori_loop(0, STAGE_ROWS, _zrow, 0)
    for t in range(ROWS_PER_SUBCORE // STAGE_ROWS):
        start = sid * ROWS_PER_SUBCORE + t * STAGE_ROWS
        pltpu.sync_copy(p0, acc.at[pl.ds(start, STAGE_ROWS)])
    plsc.subcore_barrier()

    def _idx_dst(i):
        return idxb.at[pl.ds(lax.rem(i, 4) * 2, 2)]

    def _load_idx(i, sem_i):
        pltpu.async_copy(idx_hbm.at[chunk0 + i], _idx_dst(i), sem_i)

    def _wait_idx(i, sem_i):
        pltpu.make_async_copy(idx_hbm.at[0], _idx_dst(i), sem_i).wait()

    def _start_cx(i, c_b, x_b, sem_c, sem_x):
        pltpu.async_copy(c_hbm.at[pl.ds((chunk0 + i) * CH, CH)], c_b, sem_c)
        pltpu.async_copy(x1_hbm.at[idxb.at[lax.rem(i, 4) * 2]], x_b, sem_x)

    def _step(i, bufs, obufs):
        c_b, x_b, p_b, sem_c, sem_x, sem_s, sem_i = bufs
        _, _, po_b, _, _, sem_so, _ = obufs
        pltpu.make_async_copy(c_hbm.at[pl.ds(0, CH)], c_b, sem_c).wait()
        pltpu.make_async_copy(x1_hbm.at[idxb.at[0]], x_b, sem_x).wait()

        hi_mask = jnp.full((16,), -65536, jnp.int32)

        @plsc.parallel_loop(0, CH, unroll=4)
        def _row(j):
            for k in range(D // 32):
                cw = c_b[j, pl.ds(k * 16, 16)]
                c_lo = lax.bitcast_convert_type(
                    lax.shift_left(cw, 16), jnp.float32)
                c_hi = lax.bitcast_convert_type(
                    jnp.bitwise_and(cw, hi_mask), jnp.float32)
                p_b[j, pl.ds(k * 32, 16)] = c_lo * x_b[j, pl.ds(k * 32, 16)]
                p_b[j, pl.ds(k * 32 + 16, 16)] = (
                    c_hi * x_b[j, pl.ds(k * 32 + 16, 16)])

        @pl.when(i > 0)
        def _():
            pltpu.make_async_copy(po_b, acc.at[idxb.at[1]], sem_so).wait()
        pltpu.async_copy(p_b, acc.at[idxb.at[lax.rem(i, 4) * 2 + 1]], sem_s,
                         add=True)

        @pl.when(i + 2 < N_CHUNKS)
        def _():
            _wait_idx(i + 2, sem_i)
            _start_cx(i + 2, c_b, x_b, sem_c, sem_x)

        @pl.when(i + 4 < N_CHUNKS)
        def _():
            _load_idx(i + 4, sem_i)

    bufs0 = (c0, x0, p0, sem_c0, sem_x0, sem_s0, sem_i0)
    bufs1 = (c1, x1b, p1, sem_c1, sem_x1, sem_s1, sem_i1)

    pltpu.sync_copy(idx_hbm.at[chunk0], _idx_dst(0))
    pltpu.sync_copy(idx_hbm.at[chunk0 + 1], _idx_dst(1))
    _load_idx(2, sem_i0)
    _load_idx(3, sem_i1)
    _start_cx(0, c0, x0, sem_c0, sem_x0)
    _start_cx(1, c1, x1b, sem_c1, sem_x1)

    def _pair(t, carry):
        i = 2 * t
        _step(i, bufs0, bufs1)
        _step(i + 1, bufs1, bufs0)
        return carry
    lax.fori_loop(0, N_CHUNKS // 2, _pair, 0)
    pltpu.make_async_copy(p1, acc.at[idxb.at[1]], sem_s1).wait()

    plsc.subcore_barrier()
    for t in range(ROWS_PER_SUBCORE // STAGE_ROWS):
        start = sid * ROWS_PER_SUBCORE + t * STAGE_ROWS
        pltpu.sync_copy(acc.at[pl.ds(start, STAGE_ROWS)], p0)
        pltpu.sync_copy(p0, out_hbm.at[pl.ds(cid * N_PAD + start, STAGE_ROWS)])


@functools.partial(
    pl.kernel,
    out_type=jax.ShapeDtypeStruct((2 * N_PAD, D), jnp.float32),
    mesh=plsc.VectorSubcoreMesh(core_axis_name="c", subcore_axis_name="s"),
    scratch_types=[
        pltpu.VMEM((8, CH), jnp.int32),
        pltpu.VMEM((CH, D // 2), jnp.int32),
        pltpu.VMEM((CH, D // 2), jnp.int32),
        pltpu.VMEM((CH, D), jnp.float32),
        pltpu.VMEM((CH, D), jnp.float32),
        pltpu.VMEM((CH, D), jnp.float32),
        pltpu.VMEM((CH, D), jnp.float32),
        pltpu.VMEM_SHARED((N_PAD, D), jnp.float32),
        pltpu.SemaphoreType.DMA,
        pltpu.SemaphoreType.DMA,
        pltpu.SemaphoreType.DMA,
        pltpu.SemaphoreType.DMA,
        pltpu.SemaphoreType.DMA,
        pltpu.SemaphoreType.DMA,
        pltpu.SemaphoreType.DMA,
        pltpu.SemaphoreType.DMA,
    ],
)
def _sc_aggregate(idx_hbm, c_hbm, x1_hbm, out_hbm,
                  idxb, c0, c1, x0, x1b, p0, p1, acc,
                  sem_c0, sem_c1, sem_x0, sem_x1, sem_s0, sem_s1,
                  sem_i0, sem_i1):
    _sc_agg_body(idx_hbm, c_hbm, x1_hbm, out_hbm,
                 idxb, c0, c1, x0, x1b, p0, p1, acc,
                 sem_c0, sem_c1, sem_x0, sem_x1, sem_s0, sem_s1,
                 sem_i0, sem_i1)


def _post_body(p_ref, x_ref, attr_ref, w2_ref, wsc_ref, o_ref):
    agg = p_ref[0] + p_ref[1]
    y = jnp.dot(agg, w2_ref[...],
                preferred_element_type=jnp.float32) * _POST_SCALE
    for v in range(A):
        y = y + jnp.dot(x_ref[...] * attr_ref[:, v:v + 1], wsc_ref[v],
                        preferred_element_type=jnp.float32) * _SC_SCALE
    o_ref[...] = y


def _post(partial, x, attr, w2, wsc_t):
    blk = 2000
    return pl.pallas_call(
        _post_body,
        grid=(N // blk,),
        in_specs=[
            pl.BlockSpec((2, blk, D), lambda i: (0, i, 0)),
            pl.BlockSpec((blk, D), lambda i: (i, 0)),
            pl.BlockSpec((blk, A), lambda i: (i, 0)),
            pl.BlockSpec((D, D), lambda i: (0, 0)),
            pl.BlockSpec((A, D, D), lambda i: (0, 0, 0)),
        ],
        out_specs=pl.BlockSpec((blk, D), lambda i: (i, 0)),
        out_shape=jax.ShapeDtypeStruct((N, D), jnp.float32),
    )(partial, x, attr, w2, wsc_t)


def kernel(node_features, node_attr, edge_attr, edge_embedding, edge_index,
           W_lin1, fc_W1, fc_W2, W_lin2, W_sc):
    perm = jnp.asarray(_PERM)
    w1p = jnp.pad(fc_W1, ((0, 0), (0, D - H)))
    w2e = jnp.pad(fc_W2[:, 0::2], ((0, D - H), (0, 0)))
    w2o = jnp.pad(fc_W2[:, 1::2], ((0, D - H), (0, 0)))
    c_i32, idx_pack = _edge_coef(edge_embedding, edge_attr, edge_index,
                                 w1p, w2e, w2o)
    x1 = _lin1(node_features, W_lin1[:, perm])
    partial = _sc_aggregate(idx_pack, c_i32, x1)
    partial = partial.reshape(2, N_PAD, D)[:, :N, :]
    return _post(partial, node_features, node_attr,
                 W_lin2[perm, :], W_sc.transpose(1, 0, 2))


# speedup vs baseline: 1.0276x; 1.0247x over previous
import functools

import jax
import jax.numpy as jnp
import numpy as np
from jax import lax
from jax.experimental import pallas as pl
from jax.experimental.pallas import tpu as pltpu
from jax.experimental.pallas import tpu_sc as plsc

N = 10000
E = 320000
D = 128
A = 4
B = 8
H = 8

NUM_WORKERS = 32
CH = 40
N_CHUNKS = E // (NUM_WORKERS * CH)
EDGES_PER_WORKER = CH * N_CHUNKS
N_PAD = 10240
ROWS_PER_SUBCORE = N_PAD // 16
STAGE_ROWS = 40

_INV_SQRT_B = 1.0 / np.sqrt(float(B))
_INV_SQRT_H = 1.0 / np.sqrt(float(H))
_INV_SQRT_D = 1.0 / np.sqrt(float(D))
_POST_SCALE = 1.0 / (np.sqrt(32.0) * np.sqrt(float(D)))
_SC_SCALE = 1.0 / np.sqrt(float(D * A))

_PERM = np.empty((D,), dtype=np.int32)
for _blk in range(D // 32):
    for _half in range(2):
        for _m in range(16):
            _PERM[_blk * 32 + _half * 16 + _m] = _blk * 32 + 2 * _m + _half


def _edge_coef_body(emb_ref, ea_ref, ei_ref, w1_ref, w2e_ref, w2o_ref,
                    out_ref, idx_ref):
    h = jnp.dot(emb_ref[...], w1_ref[...],
                preferred_element_type=jnp.float32) * _INV_SQRT_B
    h = h * jax.nn.sigmoid(h)
    h = h * (ea_ref[...] * _INV_SQRT_H)
    we = jnp.dot(h, w2e_ref[...], preferred_element_type=jnp.float32)
    wo = jnp.dot(h, w2o_ref[...], preferred_element_type=jnp.float32)
    pe = lax.convert_element_type(
        lax.bitcast_convert_type(we.astype(jnp.bfloat16), jnp.uint16),
        jnp.int32)
    po = lax.convert_element_type(
        lax.bitcast_convert_type(wo.astype(jnp.bfloat16), jnp.uint16),
        jnp.int32)
    out_ref[...] = jnp.bitwise_or(pe, lax.shift_left(po, 16))
    idx_ref[...] = ei_ref[...].reshape(2, _EBLK // CH, CH).swapaxes(0, 1)


_EBLK = 3200


def _edge_coef(emb, ea, ei, w1, w2e, w2o):
    grid = E // _EBLK
    return pl.pallas_call(
        _edge_coef_body,
        grid=(grid,),
        in_specs=[
            pl.BlockSpec((_EBLK, B), lambda i: (i, 0)),
            pl.BlockSpec((_EBLK, 1), lambda i: (i, 0)),
            pl.BlockSpec((2, _EBLK), lambda i: (0, i)),
            pl.BlockSpec((B, D), lambda i: (0, 0)),
            pl.BlockSpec((D, D // 2), lambda i: (0, 0)),
            pl.BlockSpec((D, D // 2), lambda i: (0, 0)),
        ],
        out_specs=[
            pl.BlockSpec((_EBLK, D // 2), lambda i: (i, 0)),
            pl.BlockSpec((_EBLK // CH, 2, CH), lambda i: (i, 0, 0)),
        ],
        out_shape=[
            jax.ShapeDtypeStruct((E, D // 2), jnp.int32),
            jax.ShapeDtypeStruct((E // CH, 2, CH), jnp.int32),
        ],
    )(emb, ea, ei, w1, w2e, w2o)


def _lin1_body(x_ref, w_ref, o_ref):
    o_ref[...] = jnp.dot(x_ref[...], w_ref[...],
                         preferred_element_type=jnp.float32) * _INV_SQRT_D


def _lin1(x, w):
    blk = 2000
    return pl.pallas_call(
        _lin1_body,
        grid=(N // blk,),
        in_specs=[
            pl.BlockSpec((blk, D), lambda i: (i, 0)),
            pl.BlockSpec((D, D), lambda i: (0, 0)),
        ],
        out_specs=pl.BlockSpec((blk, D), lambda i: (i, 0)),
        out_shape=jax.ShapeDtypeStruct((N, D), jnp.float32),
    )(x, w)


def _sc_agg_body(idx_hbm, c_hbm, x1_hbm, out_hbm,
                 idxb, c0, c1, x0, x1b, p0, p1, acc,
                 sem_c0, sem_c1, sem_x0, sem_x1, sem_s0, sem_s1,
                 sem_i0, sem_i1):
    cid = lax.axis_index("c")
    sid = lax.axis_index("s")
    wid = sid * 2 + cid
    chunk0 = wid * N_CHUNKS

    def _zrow(j, carry):
        for k in range(D // 16):
            p0[j, pl.ds(k * 16, 16)] = jnp.zeros((16,), jnp.float32)
        return carry
    lax.fori_loop(0, STAGE_ROWS, _zrow, 0)
    for t in range(ROWS_PER_SUBCORE // STAGE_ROWS):
        start = sid * ROWS_PER_SUBCORE + t * STAGE_ROWS
        pltpu.sync_copy(p0, acc.at[pl.ds(start, STAGE_ROWS)])
    plsc.subcore_barrier()

    def _idx_dst(i):
        return idxb.at[pl.ds(lax.rem(i, 4) * 2, 2)]

    def _load_idx(i, sem_i):
        pltpu.async_copy(idx_hbm.at[chunk0 + i], _idx_dst(i), sem_i)

    def _wait_idx(i, sem_i):
        pltpu.make_async_copy(idx_hbm.at[0], _idx_dst(i), sem_i).wait()

    def _start_cx(i, c_b, x_b, sem_c, sem_x):
        pltpu.async_copy(c_hbm.at[pl.ds((chunk0 + i) * CH, CH)], c_b, sem_c)
        pltpu.async_copy(x1_hbm.at[idxb.at[lax.rem(i, 4) * 2]], x_b, sem_x)

    def _step(i, bufs, obufs):
        c_b, x_b, p_b, sem_c, sem_x, sem_s, sem_i = bufs
        _, _, po_b, _, _, sem_so, _ = obufs
        pltpu.make_async_copy(c_hbm.at[pl.ds(0, CH)], c_b, sem_c).wait()
        pltpu.make_async_copy(x1_hbm.at[idxb.at[0]], x_b, sem_x).wait()

        hi_mask = jnp.full((16,), -65536, jnp.int32)

        @plsc.parallel_loop(0, CH, unroll=4)
        def _row(j):
            for k in range(D // 32):
                cw = c_b[j, pl.ds(k * 16, 16)]
                c_lo = lax.bitcast_convert_type(
                    lax.shift_left(cw, 16), jnp.float32)
                c_hi = lax.bitcast_convert_type(
                    jnp.bitwise_and(cw, hi_mask), jnp.float32)
                p_b[j, pl.ds(k * 32, 16)] = c_lo * x_b[j, pl.ds(k * 32, 16)]
                p_b[j, pl.ds(k * 32 + 16, 16)] = (
                    c_hi * x_b[j, pl.ds(k * 32 + 16, 16)])

        @pl.when(i > 0)
        def _():
            pltpu.make_async_copy(po_b, acc.at[idxb.at[1]], sem_so).wait()
        pltpu.async_copy(p_b, acc.at[idxb.at[lax.rem(i, 4) * 2 + 1]], sem_s,
                         add=True)

        @pl.when(i + 2 < N_CHUNKS)
        def _():
            _wait_idx(i + 2, sem_i)
            _start_cx(i + 2, c_b, x_b, sem_c, sem_x)

        @pl.when(i + 4 < N_CHUNKS)
        def _():
            _load_idx(i + 4, sem_i)

    bufs0 = (c0, x0, p0, sem_c0, sem_x0, sem_s0, sem_i0)
    bufs1 = (c1, x1b, p1, sem_c1, sem_x1, sem_s1, sem_i1)

    pltpu.sync_copy(idx_hbm.at[chunk0], _idx_dst(0))
    pltpu.sync_copy(idx_hbm.at[chunk0 + 1], _idx_dst(1))
    _load_idx(2, sem_i0)
    _load_idx(3, sem_i1)
    _start_cx(0, c0, x0, sem_c0, sem_x0)
    _start_cx(1, c1, x1b, sem_c1, sem_x1)

    def _pair(t, carry):
        i = 2 * t
        _step(i, bufs0, bufs1)
        _step(i + 1, bufs1, bufs0)
        return carry
    lax.fori_loop(0, N_CHUNKS // 2, _pair, 0)
    pltpu.make_async_copy(p1, acc.at[idxb.at[1]], sem_s1).wait()

    plsc.subcore_barrier()
    for t in range(ROWS_PER_SUBCORE // STAGE_ROWS):
        start = sid * ROWS_PER_SUBCORE + t * STAGE_ROWS
        pltpu.sync_copy(acc.at[pl.ds(start, STAGE_ROWS)], p0)
        pltpu.sync_copy(p0, out_hbm.at[pl.ds(cid * N_PAD + start, STAGE_ROWS)])


@functools.partial(
    pl.kernel,
    out_type=jax.ShapeDtypeStruct((2 * N_PAD, D), jnp.float32),
    mesh=plsc.VectorSubcoreMesh(core_axis_name="c", subcore_axis_name="s"),
    scratch_types=[
        pltpu.VMEM((8, CH), jnp.int32),
        pltpu.VMEM((CH, D // 2), jnp.int32),
        pltpu.VMEM((CH, D // 2), jnp.int32),
        pltpu.VMEM((CH, D), jnp.float32),
        pltpu.VMEM((CH, D), jnp.float32),
        pltpu.VMEM((CH, D), jnp.float32),
        pltpu.VMEM((CH, D), jnp.float32),
        pltpu.VMEM_SHARED((N_PAD, D), jnp.float32),
        pltpu.SemaphoreType.DMA,
        pltpu.SemaphoreType.DMA,
        pltpu.SemaphoreType.DMA,
        pltpu.SemaphoreType.DMA,
        pltpu.SemaphoreType.DMA,
        pltpu.SemaphoreType.DMA,
        pltpu.SemaphoreType.DMA,
        pltpu.SemaphoreType.DMA,
    ],
)
def _sc_aggregate(idx_hbm, c_hbm, x1_hbm, out_hbm,
                  idxb, c0, c1, x0, x1b, p0, p1, acc,
                  sem_c0, sem_c1, sem_x0, sem_x1, sem_s0, sem_s1,
                  sem_i0, sem_i1):
    _sc_agg_body(idx_hbm, c_hbm, x1_hbm, out_hbm,
                 idxb, c0, c1, x0, x1b, p0, p1, acc,
                 sem_c0, sem_c1, sem_x0, sem_x1, sem_s0, sem_s1,
                 sem_i0, sem_i1)


def _post_body(p_ref, x_ref, attr_ref, w2_ref, wsc_ref, o_ref):
    agg = p_ref[0] + p_ref[1]
    y = jnp.dot(agg, w2_ref[...],
                preferred_element_type=jnp.float32) * _POST_SCALE
    for v in range(A):
        y = y + jnp.dot(x_ref[...] * attr_ref[:, v:v + 1], wsc_ref[v],
                        preferred_element_type=jnp.float32) * _SC_SCALE
    o_ref[...] = y


def _post(partial, x, attr, w2, wsc_t):
    blk = 2000
    return pl.pallas_call(
        _post_body,
        grid=(N // blk,),
        in_specs=[
            pl.BlockSpec((2, blk, D), lambda i: (0, i, 0)),
            pl.BlockSpec((blk, D), lambda i: (i, 0)),
            pl.BlockSpec((blk, A), lambda i: (i, 0)),
            pl.BlockSpec((D, D), lambda i: (0, 0)),
            pl.BlockSpec((A, D, D), lambda i: (0, 0, 0)),
        ],
        out_specs=pl.BlockSpec((blk, D), lambda i: (i, 0)),
        out_shape=jax.ShapeDtypeStruct((N, D), jnp.float32),
    )(partial, x, attr, w2, wsc_t)


def kernel(node_features, node_attr, edge_attr, edge_embedding, edge_index,
           W_lin1, fc_W1, fc_W2, W_lin2, W_sc):
    perm = jnp.asarray(_PERM)
    w1p = jnp.pad(fc_W1, ((0, 0), (0, D - H)))
    w2e = jnp.pad(fc_W2[:, 0::2], ((0, D - H), (0, 0)))
    w2o = jnp.pad(fc_W2[:, 1::2], ((0, D - H), (0, 0)))
    c_i32, idx_pack = _edge_coef(edge_embedding, edge_attr, edge_index,
                                 w1p, w2e, w2o)
    x1 = _lin1(node_features, W_lin1[:, perm])
    partial = _sc_aggregate(idx_pack, c_i32, x1)
    partial = partial.reshape(2, N_PAD, D)[:, :N, :]
    return _post(partial, node_features, node_attr,
                 W_lin2[perm, :], W_sc.transpose(1, 0, 2))


# speedup vs baseline: 1.0807x; 1.0517x over previous
import functools

import jax
import jax.numpy as jnp
import numpy as np
from jax import lax
from jax.experimental import pallas as pl
from jax.experimental.pallas import tpu as pltpu
from jax.experimental.pallas import tpu_sc as plsc

N = 10000
E = 320000
D = 128
A = 4
B = 8
H = 8

NUM_WORKERS = 32
CH = 40
N_CHUNKS = E // (NUM_WORKERS * CH)
EDGES_PER_WORKER = CH * N_CHUNKS
N_PAD = 10240
ROWS_PER_SUBCORE = N_PAD // 16
STAGE_ROWS = 40

_INV_SQRT_B = 1.0 / np.sqrt(float(B))
_INV_SQRT_H = 1.0 / np.sqrt(float(H))
_INV_SQRT_D = 1.0 / np.sqrt(float(D))
_POST_SCALE = 1.0 / (np.sqrt(32.0) * np.sqrt(float(D)))
_SC_SCALE = 1.0 / np.sqrt(float(D * A))

_PERM = np.empty((D,), dtype=np.int32)
for _blk in range(D // 32):
    for _half in range(2):
        for _m in range(16):
            _PERM[_blk * 32 + _half * 16 + _m] = _blk * 32 + 2 * _m + _half


def _edge_coef_body(emb_ref, ea_ref, ei_ref, w1_ref, w2e_ref, w2o_ref,
                    out_ref, idx_ref):
    h = jnp.dot(emb_ref[...], w1_ref[...],
                preferred_element_type=jnp.float32) * _INV_SQRT_B
    h = h * jax.nn.sigmoid(h)
    h = h * (ea_ref[...] * _INV_SQRT_H)
    we = jnp.dot(h, w2e_ref[...], preferred_element_type=jnp.float32)
    wo = jnp.dot(h, w2o_ref[...], preferred_element_type=jnp.float32)
    pe = lax.convert_element_type(
        lax.bitcast_convert_type(we.astype(jnp.bfloat16), jnp.uint16),
        jnp.int32)
    po = lax.convert_element_type(
        lax.bitcast_convert_type(wo.astype(jnp.bfloat16), jnp.uint16),
        jnp.int32)
    out_ref[...] = jnp.bitwise_or(pe, lax.shift_left(po, 16))
    idx_ref[...] = ei_ref[...].reshape(2, _EBLK // CH, CH).swapaxes(0, 1)


_EBLK = 6400


def _edge_coef(emb, ea, ei, w1, w2e, w2o):
    grid = E // _EBLK
    return pl.pallas_call(
        _edge_coef_body,
        grid=(grid,),
        in_specs=[
            pl.BlockSpec((_EBLK, B), lambda i: (i, 0)),
            pl.BlockSpec((_EBLK, 1), lambda i: (i, 0)),
            pl.BlockSpec((2, _EBLK), lambda i: (0, i)),
            pl.BlockSpec((B, D), lambda i: (0, 0)),
            pl.BlockSpec((D, D // 2), lambda i: (0, 0)),
            pl.BlockSpec((D, D // 2), lambda i: (0, 0)),
        ],
        out_specs=[
            pl.BlockSpec((_EBLK, D // 2), lambda i: (i, 0)),
            pl.BlockSpec((_EBLK // CH, 2, CH), lambda i: (i, 0, 0)),
        ],
        out_shape=[
            jax.ShapeDtypeStruct((E, D // 2), jnp.int32),
            jax.ShapeDtypeStruct((E // CH, 2, CH), jnp.int32),
        ],
    )(emb, ea, ei, w1, w2e, w2o)


def _lin1_body(x_ref, w_ref, o_ref):
    o_ref[...] = jnp.dot(x_ref[...], w_ref[...],
                         preferred_element_type=jnp.float32) * _INV_SQRT_D


def _lin1(x, w):
    blk = 2000
    return pl.pallas_call(
        _lin1_body,
        grid=(N // blk,),
        in_specs=[
            pl.BlockSpec((blk, D), lambda i: (i, 0)),
            pl.BlockSpec((D, D), lambda i: (0, 0)),
        ],
        out_specs=pl.BlockSpec((blk, D), lambda i: (i, 0)),
        out_shape=jax.ShapeDtypeStruct((N, D), jnp.float32),
    )(x, w)


def _sc_agg_body(idx_hbm, c_hbm, x1_hbm, out_hbm,
                 idxb, c0, c1, x0, x1b, p0, p1, acc,
                 sem_c0, sem_c1, sem_x0, sem_x1, sem_s0, sem_s1,
                 sem_i0, sem_i1):
    cid = lax.axis_index("c")
    sid = lax.axis_index("s")
    wid = sid * 2 + cid
    chunk0 = wid * N_CHUNKS

    def _zrow(j, carry):
        for k in range(D // 16):
            p0[j, pl.ds(k * 16, 16)] = jnp.zeros((16,), jnp.float32)
        return carry
    lax.fori_loop(0, STAGE_ROWS, _zrow, 0)
    for t in range(ROWS_PER_SUBCORE // STAGE_ROWS):
        start = sid * ROWS_PER_SUBCORE + t * STAGE_ROWS
        pltpu.sync_copy(p0, acc.at[pl.ds(start, STAGE_ROWS)])
    plsc.subcore_barrier()

    def _idx_dst(i):
        return idxb.at[pl.ds(lax.rem(i, 4) * 2, 2)]

    def _load_idx(i, sem_i):
        pltpu.async_copy(idx_hbm.at[chunk0 + i], _idx_dst(i), sem_i)

    def _wait_idx(i, sem_i):
        pltpu.make_async_copy(idx_hbm.at[0], _idx_dst(i), sem_i).wait()

    def _start_cx(i, c_b, x_b, sem_c, sem_x):
        pltpu.async_copy(c_hbm.at[pl.ds((chunk0 + i) * CH, CH)], c_b, sem_c)
        pltpu.async_copy(x1_hbm.at[idxb.at[lax.rem(i, 4) * 2]], x_b, sem_x)

    def _step(i, bufs, obufs):
        c_b, x_b, p_b, sem_c, sem_x, sem_s, sem_i = bufs
        _, _, po_b, _, _, sem_so, _ = obufs
        pltpu.make_async_copy(c_hbm.at[pl.ds(0, CH)], c_b, sem_c).wait()
        pltpu.make_async_copy(x1_hbm.at[idxb.at[0]], x_b, sem_x).wait()

        hi_mask = jnp.full((16,), -65536, jnp.int32)

        @plsc.parallel_loop(0, CH, unroll=4)
        def _row(j):
            for k in range(D // 32):
                cw = c_b[j, pl.ds(k * 16, 16)]
                c_lo = lax.bitcast_convert_type(
                    lax.shift_left(cw, 16), jnp.float32)
                c_hi = lax.bitcast_convert_type(
                    jnp.bitwise_and(cw, hi_mask), jnp.float32)
                p_b[j, pl.ds(k * 32, 16)] = c_lo * x_b[j, pl.ds(k * 32, 16)]
                p_b[j, pl.ds(k * 32 + 16, 16)] = (
                    c_hi * x_b[j, pl.ds(k * 32 + 16, 16)])

        @pl.when(i > 0)
        def _():
            pltpu.make_async_copy(po_b, acc.at[idxb.at[1]], sem_so).wait()
        pltpu.async_copy(p_b, acc.at[idxb.at[lax.rem(i, 4) * 2 + 1]], sem_s,
                         add=True)

        @pl.when(i + 2 < N_CHUNKS)
        def _():
            _wait_idx(i + 2, sem_i)
            _start_cx(i + 2, c_b, x_b, sem_c, sem_x)

        @pl.when(i + 4 < N_CHUNKS)
        def _():
            _load_idx(i + 4, sem_i)

    bufs0 = (c0, x0, p0, sem_c0, sem_x0, sem_s0, sem_i0)
    bufs1 = (c1, x1b, p1, sem_c1, sem_x1, sem_s1, sem_i1)

    pltpu.sync_copy(idx_hbm.at[chunk0], _idx_dst(0))
    pltpu.sync_copy(idx_hbm.at[chunk0 + 1], _idx_dst(1))
    _load_idx(2, sem_i0)
    _load_idx(3, sem_i1)
    _start_cx(0, c0, x0, sem_c0, sem_x0)
    _start_cx(1, c1, x1b, sem_c1, sem_x1)

    def _pair(t, carry):
        i = 2 * t
        _step(i, bufs0, bufs1)
        _step(i + 1, bufs1, bufs0)
        return carry
    lax.fori_loop(0, N_CHUNKS // 2, _pair, 0)
    pltpu.make_async_copy(p1, acc.at[idxb.at[1]], sem_s1).wait()

    plsc.subcore_barrier()
    for t in range(ROWS_PER_SUBCORE // STAGE_ROWS):
        start = sid * ROWS_PER_SUBCORE + t * STAGE_ROWS
        pltpu.sync_copy(acc.at[pl.ds(start, STAGE_ROWS)], p0)
        pltpu.sync_copy(p0, out_hbm.at[pl.ds(cid * N_PAD + start, STAGE_ROWS)])


@functools.partial(
    pl.kernel,
    out_type=jax.ShapeDtypeStruct((2 * N_PAD, D), jnp.float32),
    mesh=plsc.VectorSubcoreMesh(core_axis_name="c", subcore_axis_name="s"),
    scratch_types=[
        pltpu.VMEM((8, CH), jnp.int32),
        pltpu.VMEM((CH, D // 2), jnp.int32),
        pltpu.VMEM((CH, D // 2), jnp.int32),
        pltpu.VMEM((CH, D), jnp.float32),
        pltpu.VMEM((CH, D), jnp.float32),
        pltpu.VMEM((CH, D), jnp.float32),
        pltpu.VMEM((CH, D), jnp.float32),
        pltpu.VMEM_SHARED((N_PAD, D), jnp.float32),
        pltpu.SemaphoreType.DMA,
        pltpu.SemaphoreType.DMA,
        pltpu.SemaphoreType.DMA,
        pltpu.SemaphoreType.DMA,
        pltpu.SemaphoreType.DMA,
        pltpu.SemaphoreType.DMA,
        pltpu.SemaphoreType.DMA,
        pltpu.SemaphoreType.DMA,
    ],
)
def _sc_aggregate(idx_hbm, c_hbm, x1_hbm, out_hbm,
                  idxb, c0, c1, x0, x1b, p0, p1, acc,
                  sem_c0, sem_c1, sem_x0, sem_x1, sem_s0, sem_s1,
                  sem_i0, sem_i1):
    _sc_agg_body(idx_hbm, c_hbm, x1_hbm, out_hbm,
                 idxb, c0, c1, x0, x1b, p0, p1, acc,
                 sem_c0, sem_c1, sem_x0, sem_x1, sem_s0, sem_s1,
                 sem_i0, sem_i1)


def _post_body(p_ref, x_ref, attr_ref, w2_ref, wsc_ref, o_ref):
    agg = p_ref[0] + p_ref[1]
    y = jnp.dot(agg, w2_ref[...],
                preferred_element_type=jnp.float32) * _POST_SCALE
    for v in range(A):
        y = y + jnp.dot(x_ref[...] * attr_ref[:, v:v + 1], wsc_ref[v],
                        preferred_element_type=jnp.float32) * _SC_SCALE
    o_ref[...] = y


def _post(partial, x, attr, w2, wsc_t):
    blk = 2000
    return pl.pallas_call(
        _post_body,
        grid=(N // blk,),
        in_specs=[
            pl.BlockSpec((2, blk, D), lambda i: (0, i, 0)),
            pl.BlockSpec((blk, D), lambda i: (i, 0)),
            pl.BlockSpec((blk, A), lambda i: (i, 0)),
            pl.BlockSpec((D, D), lambda i: (0, 0)),
            pl.BlockSpec((A, D, D), lambda i: (0, 0, 0)),
        ],
        out_specs=pl.BlockSpec((blk, D), lambda i: (i, 0)),
        out_shape=jax.ShapeDtypeStruct((N, D), jnp.float32),
    )(partial, x, attr, w2, wsc_t)


def kernel(node_features, node_attr, edge_attr, edge_embedding, edge_index,
           W_lin1, fc_W1, fc_W2, W_lin2, W_sc):
    perm = jnp.asarray(_PERM)
    w1p = jnp.pad(fc_W1, ((0, 0), (0, D - H)))
    w2e = jnp.pad(fc_W2[:, 0::2], ((0, D - H), (0, 0)))
    w2o = jnp.pad(fc_W2[:, 1::2], ((0, D - H), (0, 0)))
    c_i32, idx_pack = _edge_coef(edge_embedding, edge_attr, edge_index,
                                 w1p, w2e, w2o)
    x1 = _lin1(node_features, W_lin1[:, perm])
    partial = _sc_aggregate(idx_pack, c_i32, x1)
    partial = partial.reshape(2, N_PAD, D)[:, :N, :]
    return _post(partial, node_features, node_attr,
                 W_lin2[perm, :], W_sc.transpose(1, 0, 2))
